# baseline (speedup 1.0000x reference)
import functools

import jax
import jax.numpy as jnp
from jax import lax
from jax.experimental import pallas as pl
from jax.experimental.pallas import tpu as pltpu
from jax.experimental.pallas import tpu_sc as plsc


@functools.partial(jax.jit, static_argnums=(1,))
def _sc_copy(table, seq_len):
    d_model = table.shape[1]
    mesh = plsc.VectorSubcoreMesh(core_axis_name="c", subcore_axis_name="s")
    n_workers = 32
    rows_per_w = seq_len // n_workers

    @functools.partial(
        pl.kernel,
        mesh=mesh,
        out_type=jax.ShapeDtypeStruct((seq_len, d_model), table.dtype),
    )
    def k(table_hbm, out_hbm):
        wid = lax.axis_index("s") * 2 + lax.axis_index("c")
        base = wid * rows_per_w
        pltpu.sync_copy(
            table_hbm.at[pl.ds(base, rows_per_w), :],
            out_hbm.at[pl.ds(base, rows_per_w), :],
        )

    return k(table)


def kernel(x, position_embeddings):
    seq_len = x.shape[1]
    out = _sc_copy(position_embeddings, seq_len)
    return out[None, :, :]


# speedup vs baseline: 23.7510x; 23.7510x over previous
import functools

import jax
import jax.numpy as jnp
from jax import lax
from jax.experimental import pallas as pl
from jax.experimental.pallas import tpu as pltpu
from jax.experimental.pallas import tpu_sc as plsc


@functools.partial(jax.jit, static_argnums=(1,))
def _sc_copy(table, seq_len):
    d_model = table.shape[1]
    mesh = plsc.VectorSubcoreMesh(core_axis_name="c", subcore_axis_name="s")
    n_workers = 32
    rows_per_w = seq_len // n_workers

    chunk = 32

    @functools.partial(
        pl.kernel,
        mesh=mesh,
        out_type=jax.ShapeDtypeStruct((seq_len, d_model), table.dtype),
        scratch_types=[pltpu.VMEM((chunk, d_model), jnp.float32)],
    )
    def k(table_hbm, out_hbm, buf):
        wid = lax.axis_index("s") * 2 + lax.axis_index("c")
        base = wid * rows_per_w

        def body(i, carry):
            off = base + i * chunk
            pltpu.sync_copy(table_hbm.at[pl.ds(off, chunk), :], buf)
            pltpu.sync_copy(buf, out_hbm.at[pl.ds(off, chunk), :])
            return carry

        lax.fori_loop(0, rows_per_w // chunk, body, 0)

    return k(table)


def kernel(x, position_embeddings):
    seq_len = x.shape[1]
    out = _sc_copy(position_embeddings, seq_len)
    return out[None, :, :]


# speedup vs baseline: 24.1860x; 1.0183x over previous
import functools

import jax
import jax.numpy as jnp
from jax import lax
from jax.experimental import pallas as pl
from jax.experimental.pallas import tpu as pltpu
from jax.experimental.pallas import tpu_sc as plsc


@functools.partial(jax.jit, static_argnums=(1,))
def _sc_copy(table, seq_len):
    d_model = table.shape[1]
    mesh = plsc.VectorSubcoreMesh(core_axis_name="c", subcore_axis_name="s")
    n_workers = 32
    rows_per_w = seq_len // n_workers

    chunk = 16
    n_chunks = rows_per_w // chunk

    @functools.partial(
        pl.kernel,
        mesh=mesh,
        out_type=jax.ShapeDtypeStruct((seq_len, d_model), table.dtype),
        scratch_types=[
            pltpu.VMEM((2, chunk, d_model), jnp.float32),
            pltpu.SemaphoreType.DMA((2,)),
            pltpu.SemaphoreType.DMA((2,)),
        ],
    )
    def k(table_hbm, out_hbm, buf, isem, osem):
        wid = lax.axis_index("s") * 2 + lax.axis_index("c")
        base = wid * rows_per_w

        def start_in(i):
            b = i % 2
            return pltpu.async_copy(
                table_hbm.at[pl.ds(base + i * chunk, chunk), :],
                buf.at[b], isem.at[b])

        def start_out(i):
            b = i % 2
            return pltpu.async_copy(
                buf.at[b],
                out_hbm.at[pl.ds(base + i * chunk, chunk), :], osem.at[b])

        ins = [None] * n_chunks
        outs = [None] * n_chunks
        ins[0] = start_in(0)
        for i in range(n_chunks):
            if i >= 1:
                outs[i - 1].wait()
            if i + 1 < n_chunks:
                ins[i + 1] = start_in(i + 1)
            ins[i].wait()
            outs[i] = start_out(i)
        outs[n_chunks - 1].wait()

    return k(table)


def kernel(x, position_embeddings):
    seq_len = x.shape[1]
    out = _sc_copy(position_embeddings, seq_len)
    return out[None, :, :]


# speedup vs baseline: 49.4256x; 2.0436x over previous
import jax
import jax.numpy as jnp
from jax.experimental import pallas as pl
from jax.experimental.pallas import tpu as pltpu


def _copy_block(in_ref, o_ref):
    o_ref[...] = in_ref[...]


def kernel(x, position_embeddings):
    seq_len = x.shape[1]
    d_model = position_embeddings.shape[1]
    block = 1024
    out = pl.pallas_call(
        _copy_block,
        grid=(seq_len // block,),
        in_specs=[pl.BlockSpec((block, d_model), lambda i: (i, 0))],
        out_specs=pl.BlockSpec((block, d_model), lambda i: (i, 0)),
        out_shape=jax.ShapeDtypeStruct((seq_len, d_model), position_embeddings.dtype),
        compiler_params=pltpu.CompilerParams(
            dimension_semantics=("parallel",),
        ),
    )(position_embeddings)
    return out[None, :, :]
